# baseline (speedup 1.0000x reference)
import functools

import jax
import jax.numpy as jnp
from jax.experimental import pallas as pl

N = 10000
DIN = 128
DOUT = 128
BM = 400


def _pre_kernel(x_ref, w_ref, ws_ref, b_ref, sup_ref, self_ref):
    xb = x_ref[...]
    sup_ref[...] = jnp.dot(xb, w_ref[...], preferred_element_type=jnp.float32)
    self_ref[...] = (
        jnp.dot(xb, ws_ref[...], preferred_element_type=jnp.float32) + b_ref[...]
    )


def _main_kernel(adj_ref, sup_ref, self_ref, out_ref, sum_ref, sq_ref):
    o = (
        jnp.dot(adj_ref[...], sup_ref[...], preferred_element_type=jnp.float32)
        + self_ref[...]
    )
    out_ref[...] = o

    @pl.when(pl.program_id(0) == 0)
    def _init():
        sum_ref[...] = jnp.zeros_like(sum_ref)
        sq_ref[...] = jnp.zeros_like(sq_ref)

    sum_ref[...] += jnp.sum(o, axis=0, keepdims=True)
    sq_ref[...] += jnp.sum(o * o, axis=0, keepdims=True)


def _norm_kernel(raw_ref, sum_ref, sq_ref, gamma_ref, beta_ref, out_ref):
    mean = sum_ref[...] * (1.0 / N)
    var = sq_ref[...] * (1.0 / N) - mean * mean
    scale = gamma_ref[...] * jax.lax.rsqrt(var + 1e-5)
    shift = beta_ref[...] - mean * scale
    out_ref[...] = raw_ref[...] * scale + shift


@functools.partial(jax.jit)
def kernel(x, adj, W, W_self, b, gamma, beta):
    b2 = b.reshape(1, DOUT)
    gamma2 = gamma.reshape(1, DOUT)
    beta2 = beta.reshape(1, DOUT)

    num_blocks = N // BM

    support, self_term = pl.pallas_call(
        _pre_kernel,
        grid=(num_blocks,),
        in_specs=[
            pl.BlockSpec((BM, DIN), lambda i: (i, 0)),
            pl.BlockSpec((DIN, DOUT), lambda i: (0, 0)),
            pl.BlockSpec((DIN, DOUT), lambda i: (0, 0)),
            pl.BlockSpec((1, DOUT), lambda i: (0, 0)),
        ],
        out_specs=[
            pl.BlockSpec((BM, DOUT), lambda i: (i, 0)),
            pl.BlockSpec((BM, DOUT), lambda i: (i, 0)),
        ],
        out_shape=[
            jax.ShapeDtypeStruct((N, DOUT), jnp.float32),
            jax.ShapeDtypeStruct((N, DOUT), jnp.float32),
        ],
    )(x, W, W_self, b2)

    raw, col_sum, col_sq = pl.pallas_call(
        _main_kernel,
        grid=(num_blocks,),
        in_specs=[
            pl.BlockSpec((BM, N), lambda i: (i, 0)),
            pl.BlockSpec((N, DOUT), lambda i: (0, 0)),
            pl.BlockSpec((BM, DOUT), lambda i: (i, 0)),
        ],
        out_specs=[
            pl.BlockSpec((BM, DOUT), lambda i: (i, 0)),
            pl.BlockSpec((1, DOUT), lambda i: (0, 0)),
            pl.BlockSpec((1, DOUT), lambda i: (0, 0)),
        ],
        out_shape=[
            jax.ShapeDtypeStruct((N, DOUT), jnp.float32),
            jax.ShapeDtypeStruct((1, DOUT), jnp.float32),
            jax.ShapeDtypeStruct((1, DOUT), jnp.float32),
        ],
    )(adj, support, self_term)

    out = pl.pallas_call(
        _norm_kernel,
        grid=(num_blocks,),
        in_specs=[
            pl.BlockSpec((BM, DOUT), lambda i: (i, 0)),
            pl.BlockSpec((1, DOUT), lambda i: (0, 0)),
            pl.BlockSpec((1, DOUT), lambda i: (0, 0)),
            pl.BlockSpec((1, DOUT), lambda i: (0, 0)),
            pl.BlockSpec((1, DOUT), lambda i: (0, 0)),
        ],
        out_specs=pl.BlockSpec((BM, DOUT), lambda i: (i, 0)),
        out_shape=jax.ShapeDtypeStruct((N, DOUT), jnp.float32),
    )(raw, col_sum, col_sq, gamma2, beta2)

    return out


# speedup vs baseline: 1.3025x; 1.3025x over previous
import jax
import jax.numpy as jnp
from jax.experimental import pallas as pl
from jax.experimental.pallas import tpu as pltpu

N = 10000
DIN = 128
DOUT = 128
BM = 400
NUM_BLOCKS = N // BM


def _gcn_kernel(
    x_ref, adj_ref, w_ref, ws_ref, b_ref, gamma_ref, beta_ref,
    out_ref, sup_ref, sum_ref, sq_ref,
):
    i = pl.program_id(0)

    @pl.when(i == 0)
    def _init():
        xf = x_ref[...]
        sup_ref[...] = jnp.dot(xf, w_ref[...], preferred_element_type=jnp.float32)
        out_ref[...] = (
            jnp.dot(xf, ws_ref[...], preferred_element_type=jnp.float32)
            + b_ref[...]
        )
        sum_ref[...] = jnp.zeros_like(sum_ref)
        sq_ref[...] = jnp.zeros_like(sq_ref)

    rows = pl.ds(i * BM, BM)
    o = out_ref[rows, :] + jnp.dot(
        adj_ref[...], sup_ref[...], preferred_element_type=jnp.float32
    )
    out_ref[rows, :] = o
    sum_ref[...] += jnp.sum(o, axis=0, keepdims=True)
    sq_ref[...] += jnp.sum(o * o, axis=0, keepdims=True)

    @pl.when(i == NUM_BLOCKS - 1)
    def _normalize():
        mean = sum_ref[...] * (1.0 / N)
        var = sq_ref[...] * (1.0 / N) - mean * mean
        scale = gamma_ref[...] * jax.lax.rsqrt(var + 1e-5)
        shift = beta_ref[...] - mean * scale
        out_ref[...] = out_ref[...] * scale + shift


@jax.jit
def kernel(x, adj, W, W_self, b, gamma, beta):
    b2 = b.reshape(1, DOUT)
    gamma2 = gamma.reshape(1, DOUT)
    beta2 = beta.reshape(1, DOUT)

    out = pl.pallas_call(
        _gcn_kernel,
        grid=(NUM_BLOCKS,),
        in_specs=[
            pl.BlockSpec((N, DIN), lambda i: (0, 0)),
            pl.BlockSpec((BM, N), lambda i: (i, 0)),
            pl.BlockSpec((DIN, DOUT), lambda i: (0, 0)),
            pl.BlockSpec((DIN, DOUT), lambda i: (0, 0)),
            pl.BlockSpec((1, DOUT), lambda i: (0, 0)),
            pl.BlockSpec((1, DOUT), lambda i: (0, 0)),
            pl.BlockSpec((1, DOUT), lambda i: (0, 0)),
        ],
        out_specs=pl.BlockSpec((N, DOUT), lambda i: (0, 0)),
        out_shape=jax.ShapeDtypeStruct((N, DOUT), jnp.float32),
        scratch_shapes=[
            pltpu.VMEM((N, DIN), jnp.float32),
            pltpu.VMEM((1, DOUT), jnp.float32),
            pltpu.VMEM((1, DOUT), jnp.float32),
        ],
    )(x, adj, W, W_self, b2, gamma2, beta2)

    return out
